# baseline (speedup 1.0000x reference)
import functools

import jax
import jax.numpy as jnp
from jax import lax
from jax.experimental import pallas as pl
from jax.experimental.pallas import tpu as pltpu
from jax.experimental.pallas import tpu_sc as plsc

DIM = 64
NI = 16384
NJ = 50
B = NI * NJ
NC = 2
NS = 16
NW = NC * NS
CH = 128
NIB = NI // CH
NBLK = NJ * NIB
BPW = NBLK // NW
NBUF = 4
L = 16

_mesh = plsc.VectorSubcoreMesh(core_axis_name="c", subcore_axis_name="s")


@functools.partial(
    pl.kernel,
    mesh=_mesh,
    out_type=jax.ShapeDtypeStruct((NJ, DIM, NI), jnp.float32),
    scratch_types=[
        pltpu.VMEM((BPW, CH), jnp.int32),
        pltpu.VMEM((NBUF, CH), jnp.int32),
        pltpu.VMEM((NBUF, CH, 128), jnp.float32),
        pltpu.VMEM((NBUF, DIM, CH), jnp.float32),
    ]
    + [pltpu.SemaphoreType.DMA] * (2 * NBUF),
    compiler_params=pltpu.CompilerParams(use_tc_tiling_on_sc=True, needs_layout_passes=False),
)
def _embed(w2_hbm, xt_hbm, out_hbm, idx_v, gidx_v, rows_v, ostage_v,
           *sems):
    gsems = sems[:NBUF]
    osems = sems[NBUF:]
    wid = lax.axis_index("s") * NC + lax.axis_index("c")
    blk0 = wid * BPW

    pltpu.sync_copy(xt_hbm.at[pl.ds(blk0, BPW)], idx_v)

    def fire(k, b):
        for g in range(CH // L):
            raw = idx_v[k, pl.ds(g * L, L)]
            gidx_v[b, pl.ds(g * L, L)] = lax.shift_right_logical(raw, 1)
        pltpu.make_async_copy(
            w2_hbm.at[gidx_v.at[b]], rows_v.at[b], gsems[b]).start()

    def out_block(k):
        j = (blk0 + k) // NIB
        ib = (blk0 + k) % NIB
        return out_hbm.at[j, :, pl.ds(ib * CH, CH)]

    l_vecs = [lax.iota(jnp.int32, L) + g * L for g in range(CH // L)]

    def select(k, b):
        cols = [(idx_v[k, pl.ds(g * L, L)] & 1) * 64 for g in range(CH // L)]

        def body(c, carry):
            for g in range(CH // L):
                vals = plsc.load_gather(rows_v.at[b], [l_vecs[g], cols[g] + c])
                ostage_v[b, c, pl.ds(g * L, L)] = vals
            return carry

        lax.fori_loop(0, DIM, body, 0)

    for b in range(NBUF):
        fire(b, b)

    def outer(g, carry):
        for b in range(NBUF):
            k = g * NBUF + b
            pltpu.make_async_copy(
                w2_hbm.at[gidx_v.at[b]], rows_v.at[b], gsems[b]).wait()
            @pl.when(k >= NBUF)
            def _():
                pltpu.make_async_copy(
                    ostage_v.at[b], out_block(k - NBUF), osems[b]).wait()
            select(k, b)
            pltpu.make_async_copy(
                ostage_v.at[b], out_block(k), osems[b]).start()
            @pl.when(k + NBUF < BPW)
            def _():
                fire(k + NBUF, b)
        return carry

    lax.fori_loop(0, BPW // NBUF, outer, 0)
    for b in range(NBUF):
        k = BPW - NBUF + b
        pltpu.make_async_copy(
            ostage_v.at[b], out_block(k), osems[b]).wait()


def kernel(x, W):
    w2 = jnp.reshape(W, (W.shape[0] // 2, 128))
    xt = jnp.reshape(jnp.transpose(x.astype(jnp.int32)), (NBLK, CH))
    out = _embed(w2, xt)
    return jnp.transpose(out, (2, 0, 1))


# speedup vs baseline: 1.4556x; 1.4556x over previous
import functools

import jax
import jax.numpy as jnp
from jax import lax
from jax.experimental import pallas as pl
from jax.experimental.pallas import tpu as pltpu
from jax.experimental.pallas import tpu_sc as plsc

DIM = 64
NI = 16384
NJ = 50
B = NI * NJ
NC = 2
NS = 16
NW = NC * NS
CH = 128
NIB = NI // CH
NBLK = NJ * NIB
BPW = NBLK // NW
NBUF = 4
L = 16

_mesh = plsc.VectorSubcoreMesh(core_axis_name="c", subcore_axis_name="s")


@functools.partial(
    pl.kernel,
    mesh=_mesh,
    out_type=jax.ShapeDtypeStruct((NJ, DIM, NI), jnp.float32),
    scratch_types=[
        pltpu.VMEM((BPW, CH), jnp.int32),
        pltpu.VMEM((NBUF, CH), jnp.int32),
        pltpu.VMEM((NBUF, CH, 128), jnp.float32),
        pltpu.VMEM((NBUF, DIM, CH), jnp.float32),
    ]
    + [pltpu.SemaphoreType.DMA] * (2 * NBUF),
    compiler_params=pltpu.CompilerParams(use_tc_tiling_on_sc=True, needs_layout_passes=False),
)
def _embed(w2_hbm, xt_hbm, out_hbm, idx_v, gidx_v, rows_v, ostage_v,
           *sems):
    gsems = sems[:NBUF]
    osems = sems[NBUF:]
    wid = lax.axis_index("s") * NC + lax.axis_index("c")
    blk0 = wid * BPW

    pltpu.sync_copy(xt_hbm.at[pl.ds(blk0, BPW)], idx_v)

    def fire(k, b):
        for g in range(CH // L):
            raw = idx_v[k, pl.ds(g * L, L)]
            gidx_v[b, pl.ds(g * L, L)] = lax.shift_right_logical(raw, 1)
        pltpu.make_async_copy(
            w2_hbm.at[gidx_v.at[b]], rows_v.at[b], gsems[b]).start()

    def out_block(k):
        j = (blk0 + k) // NIB
        ib = (blk0 + k) % NIB
        return out_hbm.at[j, :, pl.ds(ib * CH, CH)]

    l_vecs = [lax.iota(jnp.int32, L) + g * L for g in range(CH // L)]

    def select(k, b):
        cols = [(idx_v[k, pl.ds(g * L, L)] & 1) * 64 for g in range(CH // L)]

        @plsc.parallel_loop(0, DIM, unroll=4)
        def _(c):
            for g in range(CH // L):
                vals = plsc.load_gather(rows_v.at[b], [l_vecs[g], cols[g] + c])
                ostage_v[b, c, pl.ds(g * L, L)] = vals

    for b in range(NBUF):
        fire(b, b)

    def outer(g, carry):
        for b in range(NBUF):
            k = g * NBUF + b
            pltpu.make_async_copy(
                w2_hbm.at[gidx_v.at[b]], rows_v.at[b], gsems[b]).wait()
            @pl.when(k >= NBUF)
            def _():
                pltpu.make_async_copy(
                    ostage_v.at[b], out_block(k - NBUF), osems[b]).wait()
            select(k, b)
            pltpu.make_async_copy(
                ostage_v.at[b], out_block(k), osems[b]).start()
            @pl.when(k + NBUF < BPW)
            def _():
                fire(k + NBUF, b)
        return carry

    lax.fori_loop(0, BPW // NBUF, outer, 0)
    for b in range(NBUF):
        k = BPW - NBUF + b
        pltpu.make_async_copy(
            ostage_v.at[b], out_block(k), osems[b]).wait()


def kernel(x, W):
    w2 = jnp.reshape(W, (W.shape[0] // 2, 128))
    xt = jnp.reshape(jnp.transpose(x.astype(jnp.int32)), (NBLK, CH))
    out = _embed(w2, xt)
    return jnp.transpose(out, (2, 0, 1))
